# baseline (speedup 1.0000x reference)
import functools

import jax
import jax.numpy as jnp
from jax import lax
from jax.experimental import pallas as pl
from jax.experimental.pallas import tpu as pltpu
from jax.experimental.pallas import tpu_sc as plsc

T = 2048
H = 1024
I = 512
E = 8
Z = 2
NE = E + Z
K = 2
LANES = 128
TBLK = 256
NEG = -1e30

A_N = T * K
PBLK = 256
NBLK = 24
P_ALLOC = NBLK * PBLK
DUMP = P_ALLOC - PBLK

NC, NS = 2, 16
NW = NC * NS
XCH = P_ALLOC // NW
XSUB = 48
DCH = A_N // NW
DSUB = 32


def _router_body(x_ref, rw_ref, bias_ref, sel_ref, swt_ref):
    x = x_ref[...]
    logits = lax.dot_general(x, rw_ref[...], (((1,), (1,)), ((), ())),
                             preferred_element_type=jnp.float32)
    col = lax.broadcasted_iota(jnp.int32, (TBLK, LANES), 1)
    valid = col < NE
    logits = jnp.where(valid, logits, NEG)
    m = jnp.max(logits, axis=1, keepdims=True)
    ex = jnp.exp(logits - m)
    ex = jnp.where(valid, ex, 0.0)
    scores = ex / jnp.sum(ex, axis=1, keepdims=True)
    biased = jnp.where(valid, scores + bias_ref[...], NEG)

    m1 = jnp.max(biased, axis=1, keepdims=True)
    idx1 = jnp.min(jnp.where(biased == m1, col, LANES), axis=1, keepdims=True)
    oh1 = col == idx1
    w1 = jnp.sum(jnp.where(oh1, scores, 0.0), axis=1, keepdims=True)
    b2 = jnp.where(oh1, NEG, biased)
    m2 = jnp.max(b2, axis=1, keepdims=True)
    idx2 = jnp.min(jnp.where(b2 == m2, col, LANES), axis=1, keepdims=True)
    oh2 = col == idx2
    w2 = jnp.sum(jnp.where(oh2, scores, 0.0), axis=1, keepdims=True)

    zw = jnp.where(idx1 >= E, w1, 0.0) + jnp.where(idx2 >= E, w2, 0.0)
    sel_ref[...] = jnp.where(col == 0, idx1,
                             jnp.where(col == 1, idx2, 0))
    swt_ref[...] = jnp.where(col == 0, w1,
                             jnp.where(col == 1, w2,
                                       jnp.where(col == 2, zw, 0.0)))


def _dispatch_body(eid_hbm, tok_hbm, pos_hbm, meta_hbm,
                   eid_v, tok_v, pos_v, meta_v):
    cid = lax.axis_index("c")
    sid = lax.axis_index("s")

    @pl.when(jnp.logical_and(cid == 0, sid == 0))
    def _():
        pltpu.sync_copy(eid_hbm, eid_v)
        lanes = lax.iota(jnp.int32, 16)
        zero16 = jnp.zeros((16,), jnp.int32)

        def zero_body(i, c):
            tok_v[pl.ds(i * 16, 16)] = zero16
            return c
        lax.fori_loop(0, P_ALLOC // 16, zero_body, 0)

        def hist_body(g, hist):
            v = eid_v[pl.ds(g * 16, 16)]
            for e in range(E):
                s = jnp.sum(jnp.where(v == e, 1, 0))
                hist = hist + jnp.where(lanes == e, s, 0)
            return hist
        hist = lax.fori_loop(0, A_N // 16, hist_body, zero16)

        npad = jnp.where(lanes < E,
                         ((hist + (PBLK - 1)) // PBLK) * PBLK, 0)
        incl = jnp.cumsum(npad)
        offs = incl - npad
        pt = jnp.sum(jnp.where(lanes < E, npad, 0))
        cnt0 = jnp.where(lanes < E, offs, 0)

        def scan_body(g, cnt):
            v = eid_v[pl.ds(g * 16, 16)]
            bases = cnt.at[v].get(mode="promise_in_bounds")
            r = zero16
            for e in range(E):
                mk = v == e
                mi = jnp.where(mk, 1, 0)
                c = jnp.cumsum(mi)
                r = jnp.where(mk, c - 1, r)
                cnt = cnt + jnp.where(lanes == e, jnp.sum(mi), 0)
            posv = jnp.where(v < E, bases + r, DUMP)
            tv = (g * 16 + lanes) >> 1
            pos_v[pl.ds(g * 16, 16)] = posv
            plsc.store_scatter(tok_v, (posv,), tv, mask=v < E)
            return cnt
        lax.fori_loop(0, A_N // 16, scan_body, cnt0)

        lastblk = jnp.maximum(pt // PBLK - 1, 0)
        exp0 = zero16
        exp1 = zero16
        lastexp = jnp.int32(0)
        row0 = lanes * PBLK
        row1 = (lanes + 16) * PBLK
        for e in range(E):
            se = jnp.sum(jnp.where(lanes == e, incl, 0))
            exp0 = exp0 + jnp.where(row0 >= se, 1, 0)
            exp1 = exp1 + jnp.where(row1 >= se, 1, 0)
            lastexp = lastexp + jnp.where(pt - PBLK >= se, 1, 0)
        act0 = jnp.where(row0 < pt, 1, 0)
        act1 = jnp.where(row1 < pt, 1, 0)
        meta_v[pl.ds(0, 16)] = jnp.where(act0 == 1, exp0, lastexp)
        meta_v[pl.ds(16, 16)] = jnp.where(act1 == 1, exp1, lastexp)
        meta_v[pl.ds(32, 16)] = act0
        meta_v[pl.ds(48, 16)] = act1
        meta_v[pl.ds(64, 16)] = (jnp.where(lanes == 0, lastblk, 0)
                                 + jnp.where(lanes == 1, pt, 0))
        meta_v[pl.ds(80, 16)] = zero16

        pltpu.sync_copy(tok_v, tok_hbm)
        pltpu.sync_copy(pos_v, pos_hbm)
        pltpu.sync_copy(meta_v, meta_hbm)


def _xgather_body(x_hbm, tok_hbm, meta_hbm, xg_hbm,
                  idx_v, rows_a, rows_b, mvec_v, gsem_a, gsem_b,
                  wsem_a, wsem_b):
    wid = lax.axis_index("s") * NC + lax.axis_index("c")
    base = wid * XCH
    pltpu.sync_copy(meta_hbm.at[pl.ds(64, 32)], mvec_v)
    lanes = lax.iota(jnp.int32, 16)
    pt = jnp.sum(jnp.where(lanes == 1, mvec_v[pl.ds(0, 16)], 0))

    @pl.when(base < pt)
    def _():
        pltpu.sync_copy(tok_hbm.at[pl.ds(base, XCH)], idx_v)
        _pipe_gather(x_hbm, idx_v, xg_hbm, base, XCH // XSUB, XSUB,
                     (rows_a, rows_b), (gsem_a, gsem_b), (wsem_a, wsem_b))


def _pipe_gather(src_hbm, idx_v, out_hbm, base, n, sub, bufs, gsems, wsems):
    pltpu.async_copy(src_hbm.at[idx_v.at[pl.ds(0, sub)]], bufs[0], gsems[0])
    for j in range(n):
        k = j % 2
        pltpu.make_async_copy(src_hbm.at[idx_v.at[pl.ds(j * sub, sub)]],
                              bufs[k], gsems[k]).wait()
        if j + 1 < n:
            nk = (j + 1) % 2
            if j >= 1:
                pltpu.make_async_copy(bufs[nk], out_hbm.at[pl.ds(0, sub)],
                                      wsems[nk]).wait()
            pltpu.async_copy(
                src_hbm.at[idx_v.at[pl.ds((j + 1) * sub, sub)]],
                bufs[nk], gsems[nk])
        pltpu.async_copy(bufs[k], out_hbm.at[pl.ds(base + j * sub, sub)],
                         wsems[k])
    pltpu.make_async_copy(bufs[(n - 2) % 2], out_hbm.at[pl.ds(0, sub)],
                          wsems[(n - 2) % 2]).wait()
    pltpu.make_async_copy(bufs[(n - 1) % 2], out_hbm.at[pl.ds(0, sub)],
                          wsems[(n - 1) % 2]).wait()


def _group_body(meta_ref, xg_ref, w1_ref, w3_ref, w2_ref, yg_ref):
    b = pl.program_id(0)
    act = meta_ref[32 + b]

    @pl.when(act == 0)
    def _zero():
        yg_ref[...] = jnp.zeros((PBLK, H), jnp.float32)

    @pl.when(act == 1)
    def _compute():
        xb = xg_ref[...].astype(jnp.bfloat16)
        g = lax.dot_general(xb, w1_ref[0], (((1,), (1,)), ((), ())),
                            preferred_element_type=jnp.float32)
        u = lax.dot_general(xb, w3_ref[0], (((1,), (1,)), ((), ())),
                            preferred_element_type=jnp.float32)
        acts = ((g * jax.nn.sigmoid(g)) * u).astype(jnp.bfloat16)
        yg_ref[...] = lax.dot_general(
            acts, w2_ref[0], (((1,), (1,)), ((), ())),
            preferred_element_type=jnp.float32)


def _cgather_body(yg_hbm, pos_hbm, yt_hbm, idx_v, rows_a, rows_b,
                  gsem_a, gsem_b, wsem_a, wsem_b):
    wid = lax.axis_index("s") * NC + lax.axis_index("c")
    base = wid * DCH
    pltpu.sync_copy(pos_hbm.at[pl.ds(base, DCH)], idx_v)
    _pipe_gather(yg_hbm, idx_v, yt_hbm, base, DCH // DSUB, DSUB,
                 (rows_a, rows_b), (gsem_a, gsem_b), (wsem_a, wsem_b))


def _final_body(x_ref, yt_ref, swt_ref, out_ref):
    x = x_ref[...]
    yt = yt_ref[...]
    swt = swt_ref[...]
    col = lax.broadcasted_iota(jnp.int32, (TBLK, LANES), 1)
    w0 = jnp.sum(jnp.where(col == 0, swt, 0.0), axis=1, keepdims=True)
    w1 = jnp.sum(jnp.where(col == 1, swt, 0.0), axis=1, keepdims=True)
    zw = jnp.sum(jnp.where(col == 2, swt, 0.0), axis=1, keepdims=True)
    out_ref[...] = zw * x + w0 * yt[:, :H] + w1 * yt[:, H:]


_SC_MESH = plsc.VectorSubcoreMesh(core_axis_name="c", subcore_axis_name="s")


@jax.jit
def kernel(hidden_states, router_weight, e_score_correction_bias, w1, w3, w2):
    x = hidden_states.astype(jnp.float32)
    rw = jnp.zeros((LANES, H), jnp.float32).at[:NE].set(router_weight)
    bias = jnp.full((1, LANES), NEG, jnp.float32).at[0, :NE].set(
        e_score_correction_bias)
    nt = T // TBLK

    sel, swt = pl.pallas_call(
        _router_body,
        grid=(nt,),
        in_specs=[
            pl.BlockSpec((TBLK, H), lambda t: (t, 0)),
            pl.BlockSpec((LANES, H), lambda t: (0, 0)),
            pl.BlockSpec((1, LANES), lambda t: (0, 0)),
        ],
        out_specs=[
            pl.BlockSpec((TBLK, LANES), lambda t: (t, 0)),
            pl.BlockSpec((TBLK, LANES), lambda t: (t, 0)),
        ],
        out_shape=[
            jax.ShapeDtypeStruct((T, LANES), jnp.int32),
            jax.ShapeDtypeStruct((T, LANES), jnp.float32),
        ],
    )(x, rw, bias)

    eid = sel[:, :K].reshape(A_N)

    dispatch = functools.partial(
        pl.kernel,
        compiler_params=pltpu.CompilerParams(needs_layout_passes=False),
        out_type=(
            jax.ShapeDtypeStruct((P_ALLOC,), jnp.int32),
            jax.ShapeDtypeStruct((A_N,), jnp.int32),
            jax.ShapeDtypeStruct((96,), jnp.int32),
        ),
        mesh=_SC_MESH,
        scratch_types=[
            pltpu.VMEM((A_N,), jnp.int32),
            pltpu.VMEM((P_ALLOC,), jnp.int32),
            pltpu.VMEM((A_N,), jnp.int32),
            pltpu.VMEM((96,), jnp.int32),
        ],
    )(_dispatch_body)
    tok, pos, meta = dispatch(eid)

    xgather = functools.partial(
        pl.kernel,
        compiler_params=pltpu.CompilerParams(needs_layout_passes=False),
        out_type=jax.ShapeDtypeStruct((P_ALLOC, H), jnp.float32),
        mesh=_SC_MESH,
        scratch_types=[
            pltpu.VMEM((XCH,), jnp.int32),
            pltpu.VMEM((XSUB, H), jnp.float32),
            pltpu.VMEM((XSUB, H), jnp.float32),
            pltpu.VMEM((32,), jnp.int32),
            pltpu.SemaphoreType.DMA,
            pltpu.SemaphoreType.DMA,
            pltpu.SemaphoreType.DMA,
            pltpu.SemaphoreType.DMA,
        ],
    )(_xgather_body)
    xg = xgather(x, tok, meta)

    yg = pl.pallas_call(
        _group_body,
        grid_spec=pltpu.PrefetchScalarGridSpec(
            num_scalar_prefetch=1,
            grid=(NBLK,),
            in_specs=[
                pl.BlockSpec((PBLK, H),
                             lambda b, m: (jnp.where(m[32 + b] == 1, b,
                                                     m[64]), 0)),
                pl.BlockSpec((1, I, H), lambda b, m: (m[b], 0, 0)),
                pl.BlockSpec((1, I, H), lambda b, m: (m[b], 0, 0)),
                pl.BlockSpec((1, H, I), lambda b, m: (m[b], 0, 0)),
            ],
            out_specs=pl.BlockSpec((PBLK, H), lambda b, m: (b, 0)),
        ),
        out_shape=jax.ShapeDtypeStruct((P_ALLOC, H), jnp.float32),
        compiler_params=pltpu.CompilerParams(
            dimension_semantics=("arbitrary",)),
    )(meta, xg, w1.astype(jnp.bfloat16), w3.astype(jnp.bfloat16),
      w2.astype(jnp.bfloat16))

    cgather = functools.partial(
        pl.kernel,
        compiler_params=pltpu.CompilerParams(needs_layout_passes=False),
        out_type=jax.ShapeDtypeStruct((A_N, H), jnp.float32),
        mesh=_SC_MESH,
        scratch_types=[
            pltpu.VMEM((DCH,), jnp.int32),
            pltpu.VMEM((DSUB, H), jnp.float32),
            pltpu.VMEM((DSUB, H), jnp.float32),
            pltpu.SemaphoreType.DMA,
            pltpu.SemaphoreType.DMA,
            pltpu.SemaphoreType.DMA,
            pltpu.SemaphoreType.DMA,
        ],
    )(_cgather_body)
    yt = cgather(yg, pos)

    out = pl.pallas_call(
        _final_body,
        grid=(nt,),
        in_specs=[
            pl.BlockSpec((TBLK, H), lambda t: (t, 0)),
            pl.BlockSpec((TBLK, 2 * H), lambda t: (t, 0)),
            pl.BlockSpec((TBLK, LANES), lambda t: (t, 0)),
        ],
        out_specs=pl.BlockSpec((TBLK, H), lambda t: (t, 0)),
        out_shape=jax.ShapeDtypeStruct((T, H), jnp.float32),
    )(x, yt.reshape(T, K * H), swt)
    return out


# speedup vs baseline: 1.5267x; 1.5267x over previous
import functools

import jax
import jax.numpy as jnp
from jax import lax
from jax.experimental import pallas as pl
from jax.experimental.pallas import tpu as pltpu
from jax.experimental.pallas import tpu_sc as plsc

T = 2048
H = 1024
I = 512
E = 8
Z = 2
NE = E + Z
K = 2
LANES = 128
TBLK = 256
NEG = -1e30

A_N = T * K
PBLK = 256
NBLK = 24
P_ALLOC = NBLK * PBLK
DUMP = P_ALLOC - PBLK

NC, NS = 2, 16
NW = NC * NS
XCH = P_ALLOC // NW
XSUB = 48
DCH = A_N // NW
DSUB = 32


def _router_body(x_ref, rw_ref, bias_ref, sel_ref, swt_ref):
    x = x_ref[...]
    logits = lax.dot_general(x, rw_ref[...], (((1,), (1,)), ((), ())),
                             preferred_element_type=jnp.float32)
    col = lax.broadcasted_iota(jnp.int32, (TBLK, LANES), 1)
    valid = col < NE
    logits = jnp.where(valid, logits, NEG)
    m = jnp.max(logits, axis=1, keepdims=True)
    ex = jnp.exp(logits - m)
    ex = jnp.where(valid, ex, 0.0)
    scores = ex / jnp.sum(ex, axis=1, keepdims=True)
    biased = jnp.where(valid, scores + bias_ref[...], NEG)

    m1 = jnp.max(biased, axis=1, keepdims=True)
    idx1 = jnp.min(jnp.where(biased == m1, col, LANES), axis=1, keepdims=True)
    oh1 = col == idx1
    w1 = jnp.sum(jnp.where(oh1, scores, 0.0), axis=1, keepdims=True)
    b2 = jnp.where(oh1, NEG, biased)
    m2 = jnp.max(b2, axis=1, keepdims=True)
    idx2 = jnp.min(jnp.where(b2 == m2, col, LANES), axis=1, keepdims=True)
    oh2 = col == idx2
    w2 = jnp.sum(jnp.where(oh2, scores, 0.0), axis=1, keepdims=True)

    zw = jnp.where(idx1 >= E, w1, 0.0) + jnp.where(idx2 >= E, w2, 0.0)
    sel_ref[...] = jnp.where(col == 0, idx1,
                             jnp.where(col == 1, idx2, 0))
    swt_ref[...] = jnp.where(col == 0, w1,
                             jnp.where(col == 1, w2,
                                       jnp.where(col == 2, zw, 0.0)))


def _dispatch_body(eid_hbm, tok_hbm, pos_hbm, meta_hbm,
                   eid_v, tok_v, pos_v, meta_v):
    cid = lax.axis_index("c")
    sid = lax.axis_index("s")

    @pl.when(jnp.logical_and(cid == 0, sid == 0))
    def _():
        pltpu.sync_copy(eid_hbm, eid_v)
        lanes = lax.iota(jnp.int32, 16)
        zero16 = jnp.zeros((16,), jnp.int32)

        def zero_body(i, c):
            tok_v[pl.ds(i * 16, 16)] = (i * 16 + lanes) & (T - 1)
            return c
        lax.fori_loop(0, P_ALLOC // 16, zero_body, 0)

        def hist_body(g, hist):
            v = eid_v[pl.ds(g * 16, 16)]
            for e in range(E):
                s = jnp.sum(jnp.where(v == e, 1, 0))
                hist = hist + jnp.where(lanes == e, s, 0)
            return hist
        hist = lax.fori_loop(0, A_N // 16, hist_body, zero16)

        npad = jnp.where(lanes < E,
                         ((hist + (PBLK - 1)) // PBLK) * PBLK, 0)
        incl = jnp.cumsum(npad)
        offs = incl - npad
        pt = jnp.sum(jnp.where(lanes < E, npad, 0))
        cnt0 = jnp.where(lanes < E, offs, 0)

        def scan_body(g, cnt):
            v = eid_v[pl.ds(g * 16, 16)]
            bases = cnt.at[v].get(mode="promise_in_bounds")
            r = zero16
            for e in range(E):
                mk = v == e
                mi = jnp.where(mk, 1, 0)
                c = jnp.cumsum(mi)
                r = jnp.where(mk, c - 1, r)
                cnt = cnt + jnp.where(lanes == e, jnp.sum(mi), 0)
            posv = jnp.where(v < E, bases + r,
                             DUMP + ((g * 16 + lanes) & (PBLK - 1)))
            tv = (g * 16 + lanes) >> 1
            pos_v[pl.ds(g * 16, 16)] = posv
            plsc.store_scatter(tok_v, (posv,), tv, mask=v < E)
            return cnt
        lax.fori_loop(0, A_N // 16, scan_body, cnt0)

        lastblk = jnp.maximum(pt // PBLK - 1, 0)
        exp0 = zero16
        exp1 = zero16
        lastexp = jnp.int32(0)
        row0 = lanes * PBLK
        row1 = (lanes + 16) * PBLK
        for e in range(E):
            se = jnp.sum(jnp.where(lanes == e, incl, 0))
            exp0 = exp0 + jnp.where(row0 >= se, 1, 0)
            exp1 = exp1 + jnp.where(row1 >= se, 1, 0)
            lastexp = lastexp + jnp.where(pt - PBLK >= se, 1, 0)
        act0 = jnp.where(row0 < pt, 1, 0)
        act1 = jnp.where(row1 < pt, 1, 0)
        meta_v[pl.ds(0, 16)] = jnp.where(act0 == 1, exp0, lastexp)
        meta_v[pl.ds(16, 16)] = jnp.where(act1 == 1, exp1, lastexp)
        meta_v[pl.ds(32, 16)] = act0
        meta_v[pl.ds(48, 16)] = act1
        meta_v[pl.ds(64, 16)] = (jnp.where(lanes == 0, lastblk, 0)
                                 + jnp.where(lanes == 1, pt, 0))
        meta_v[pl.ds(80, 16)] = zero16

        pltpu.sync_copy(tok_v, tok_hbm)
        pltpu.sync_copy(pos_v, pos_hbm)
        pltpu.sync_copy(meta_v, meta_hbm)


def _xgather_body(x_hbm, tok_hbm, meta_hbm, xg_hbm,
                  idx_v, rows_a, rows_b, mvec_v, gsem_a, gsem_b,
                  wsem_a, wsem_b):
    wid = lax.axis_index("s") * NC + lax.axis_index("c")
    base = wid * XCH
    pltpu.sync_copy(meta_hbm.at[pl.ds(64, 32)], mvec_v)
    lanes = lax.iota(jnp.int32, 16)
    pt = jnp.sum(jnp.where(lanes == 1, mvec_v[pl.ds(0, 16)], 0))

    @pl.when(base < pt)
    def _():
        pltpu.sync_copy(tok_hbm.at[pl.ds(base, XCH)], idx_v)
        _pipe_gather(x_hbm, idx_v, xg_hbm, base, XCH // XSUB, XSUB,
                     (rows_a, rows_b), (gsem_a, gsem_b), (wsem_a, wsem_b))


def _pipe_gather(src_hbm, idx_v, out_hbm, base, n, sub, bufs, gsems, wsems):
    pltpu.async_copy(src_hbm.at[idx_v.at[pl.ds(0, sub)]], bufs[0], gsems[0])
    for j in range(n):
        k = j % 2
        pltpu.make_async_copy(src_hbm.at[idx_v.at[pl.ds(j * sub, sub)]],
                              bufs[k], gsems[k]).wait()
        if j + 1 < n:
            nk = (j + 1) % 2
            if j >= 1:
                pltpu.make_async_copy(bufs[nk], out_hbm.at[pl.ds(0, sub)],
                                      wsems[nk]).wait()
            pltpu.async_copy(
                src_hbm.at[idx_v.at[pl.ds((j + 1) * sub, sub)]],
                bufs[nk], gsems[nk])
        pltpu.async_copy(bufs[k], out_hbm.at[pl.ds(base + j * sub, sub)],
                         wsems[k])
    pltpu.make_async_copy(bufs[(n - 2) % 2], out_hbm.at[pl.ds(0, sub)],
                          wsems[(n - 2) % 2]).wait()
    pltpu.make_async_copy(bufs[(n - 1) % 2], out_hbm.at[pl.ds(0, sub)],
                          wsems[(n - 1) % 2]).wait()


def _group_body(meta_ref, xg_ref, w1_ref, w3_ref, w2_ref, yg_ref):
    b = pl.program_id(0)
    act = meta_ref[32 + b]

    @pl.when(act == 0)
    def _zero():
        yg_ref[...] = jnp.zeros((PBLK, H), jnp.float32)

    @pl.when(act == 1)
    def _compute():
        xb = xg_ref[...].astype(jnp.bfloat16)
        g = lax.dot_general(xb, w1_ref[0], (((1,), (1,)), ((), ())),
                            preferred_element_type=jnp.float32)
        u = lax.dot_general(xb, w3_ref[0], (((1,), (1,)), ((), ())),
                            preferred_element_type=jnp.float32)
        acts = ((g * jax.nn.sigmoid(g)) * u).astype(jnp.bfloat16)
        yg_ref[...] = lax.dot_general(
            acts, w2_ref[0], (((1,), (1,)), ((), ())),
            preferred_element_type=jnp.float32)


def _cgather_body(yg_hbm, pos_hbm, yt_hbm, idx_v, rows_a, rows_b,
                  gsem_a, gsem_b, wsem_a, wsem_b):
    wid = lax.axis_index("s") * NC + lax.axis_index("c")
    base = wid * DCH
    pltpu.sync_copy(pos_hbm.at[pl.ds(base, DCH)], idx_v)
    _pipe_gather(yg_hbm, idx_v, yt_hbm, base, DCH // DSUB, DSUB,
                 (rows_a, rows_b), (gsem_a, gsem_b), (wsem_a, wsem_b))


def _final_body(x_ref, yt_ref, swt_ref, out_ref):
    x = x_ref[...]
    yt = yt_ref[...]
    swt = swt_ref[...]
    col = lax.broadcasted_iota(jnp.int32, (TBLK, LANES), 1)
    w0 = jnp.sum(jnp.where(col == 0, swt, 0.0), axis=1, keepdims=True)
    w1 = jnp.sum(jnp.where(col == 1, swt, 0.0), axis=1, keepdims=True)
    zw = jnp.sum(jnp.where(col == 2, swt, 0.0), axis=1, keepdims=True)
    out_ref[...] = zw * x + w0 * yt[:, :H] + w1 * yt[:, H:]


_SC_MESH = plsc.VectorSubcoreMesh(core_axis_name="c", subcore_axis_name="s")


@jax.jit
def kernel(hidden_states, router_weight, e_score_correction_bias, w1, w3, w2):
    x = hidden_states.astype(jnp.float32)
    rw = jnp.zeros((LANES, H), jnp.float32).at[:NE].set(router_weight)
    bias = jnp.full((1, LANES), NEG, jnp.float32).at[0, :NE].set(
        e_score_correction_bias)
    nt = T // TBLK

    sel, swt = pl.pallas_call(
        _router_body,
        grid=(nt,),
        in_specs=[
            pl.BlockSpec((TBLK, H), lambda t: (t, 0)),
            pl.BlockSpec((LANES, H), lambda t: (0, 0)),
            pl.BlockSpec((1, LANES), lambda t: (0, 0)),
        ],
        out_specs=[
            pl.BlockSpec((TBLK, LANES), lambda t: (t, 0)),
            pl.BlockSpec((TBLK, LANES), lambda t: (t, 0)),
        ],
        out_shape=[
            jax.ShapeDtypeStruct((T, LANES), jnp.int32),
            jax.ShapeDtypeStruct((T, LANES), jnp.float32),
        ],
    )(x, rw, bias)

    eid = sel[:, :K].reshape(A_N)

    dispatch = functools.partial(
        pl.kernel,
        compiler_params=pltpu.CompilerParams(needs_layout_passes=False),
        out_type=(
            jax.ShapeDtypeStruct((P_ALLOC,), jnp.int32),
            jax.ShapeDtypeStruct((A_N,), jnp.int32),
            jax.ShapeDtypeStruct((96,), jnp.int32),
        ),
        mesh=_SC_MESH,
        scratch_types=[
            pltpu.VMEM((A_N,), jnp.int32),
            pltpu.VMEM((P_ALLOC,), jnp.int32),
            pltpu.VMEM((A_N,), jnp.int32),
            pltpu.VMEM((96,), jnp.int32),
        ],
    )(_dispatch_body)
    tok, pos, meta = dispatch(eid)

    xgather = functools.partial(
        pl.kernel,
        compiler_params=pltpu.CompilerParams(needs_layout_passes=False),
        out_type=jax.ShapeDtypeStruct((P_ALLOC, H), jnp.float32),
        mesh=_SC_MESH,
        scratch_types=[
            pltpu.VMEM((XCH,), jnp.int32),
            pltpu.VMEM((XSUB, H), jnp.float32),
            pltpu.VMEM((XSUB, H), jnp.float32),
            pltpu.VMEM((32,), jnp.int32),
            pltpu.SemaphoreType.DMA,
            pltpu.SemaphoreType.DMA,
            pltpu.SemaphoreType.DMA,
            pltpu.SemaphoreType.DMA,
        ],
    )(_xgather_body)
    xg = xgather(x, tok, meta)

    yg = pl.pallas_call(
        _group_body,
        grid_spec=pltpu.PrefetchScalarGridSpec(
            num_scalar_prefetch=1,
            grid=(NBLK,),
            in_specs=[
                pl.BlockSpec((PBLK, H),
                             lambda b, m: (jnp.where(m[32 + b] == 1, b,
                                                     m[64]), 0)),
                pl.BlockSpec((1, I, H), lambda b, m: (m[b], 0, 0)),
                pl.BlockSpec((1, I, H), lambda b, m: (m[b], 0, 0)),
                pl.BlockSpec((1, H, I), lambda b, m: (m[b], 0, 0)),
            ],
            out_specs=pl.BlockSpec((PBLK, H), lambda b, m: (b, 0)),
        ),
        out_shape=jax.ShapeDtypeStruct((P_ALLOC, H), jnp.float32),
        compiler_params=pltpu.CompilerParams(
            dimension_semantics=("arbitrary",)),
    )(meta, xg, w1.astype(jnp.bfloat16), w3.astype(jnp.bfloat16),
      w2.astype(jnp.bfloat16))

    cgather = functools.partial(
        pl.kernel,
        compiler_params=pltpu.CompilerParams(needs_layout_passes=False),
        out_type=jax.ShapeDtypeStruct((A_N, H), jnp.float32),
        mesh=_SC_MESH,
        scratch_types=[
            pltpu.VMEM((DCH,), jnp.int32),
            pltpu.VMEM((DSUB, H), jnp.float32),
            pltpu.VMEM((DSUB, H), jnp.float32),
            pltpu.SemaphoreType.DMA,
            pltpu.SemaphoreType.DMA,
            pltpu.SemaphoreType.DMA,
            pltpu.SemaphoreType.DMA,
        ],
    )(_cgather_body)
    yt = cgather(yg, pos)

    out = pl.pallas_call(
        _final_body,
        grid=(nt,),
        in_specs=[
            pl.BlockSpec((TBLK, H), lambda t: (t, 0)),
            pl.BlockSpec((TBLK, 2 * H), lambda t: (t, 0)),
            pl.BlockSpec((TBLK, LANES), lambda t: (t, 0)),
        ],
        out_specs=pl.BlockSpec((TBLK, H), lambda t: (t, 0)),
        out_shape=jax.ShapeDtypeStruct((T, H), jnp.float32),
    )(x, yt.reshape(T, K * H), swt)
    return out


# speedup vs baseline: 1.6235x; 1.0635x over previous
import functools

import jax
import jax.numpy as jnp
from jax import lax
from jax.experimental import pallas as pl
from jax.experimental.pallas import tpu as pltpu
from jax.experimental.pallas import tpu_sc as plsc

T = 2048
H = 1024
I = 512
E = 8
Z = 2
NE = E + Z
K = 2
LANES = 128
TBLK = 256
NEG = -1e30

A_N = T * K
PBLK = 256
NBLK = 24
P_ALLOC = NBLK * PBLK
DUMP = P_ALLOC - PBLK

NC, NS = 2, 16
NW = NC * NS
XCH = P_ALLOC // NW
XSUB = 48
DCH = A_N // NW
DSUB = 32


def _router_body(x_ref, rw_ref, bias_ref, sel_ref, swt_ref):
    x = x_ref[...]
    logits = lax.dot_general(x, rw_ref[...], (((1,), (1,)), ((), ())),
                             preferred_element_type=jnp.float32)
    col = lax.broadcasted_iota(jnp.int32, (TBLK, LANES), 1)
    valid = col < NE
    logits = jnp.where(valid, logits, NEG)
    m = jnp.max(logits, axis=1, keepdims=True)
    ex = jnp.exp(logits - m)
    ex = jnp.where(valid, ex, 0.0)
    scores = ex / jnp.sum(ex, axis=1, keepdims=True)
    biased = jnp.where(valid, scores + bias_ref[...], NEG)

    m1 = jnp.max(biased, axis=1, keepdims=True)
    idx1 = jnp.min(jnp.where(biased == m1, col, LANES), axis=1, keepdims=True)
    oh1 = col == idx1
    w1 = jnp.sum(jnp.where(oh1, scores, 0.0), axis=1, keepdims=True)
    b2 = jnp.where(oh1, NEG, biased)
    m2 = jnp.max(b2, axis=1, keepdims=True)
    idx2 = jnp.min(jnp.where(b2 == m2, col, LANES), axis=1, keepdims=True)
    oh2 = col == idx2
    w2 = jnp.sum(jnp.where(oh2, scores, 0.0), axis=1, keepdims=True)

    zw = jnp.where(idx1 >= E, w1, 0.0) + jnp.where(idx2 >= E, w2, 0.0)
    sel_ref[...] = jnp.where(col == 0, idx1,
                             jnp.where(col == 1, idx2, 0))
    swt_ref[...] = jnp.where(col == 0, w1,
                             jnp.where(col == 1, w2,
                                       jnp.where(col == 2, zw, 0.0)))


def _dispgather_body(eid_hbm, x_hbm, xg_hbm, pos_hbm, meta_hbm,
                     eid_v, tok_v, pos_v, meta_v, rows_a, rows_b,
                     gsem_a, gsem_b, wsem_a, wsem_b):
    cid = lax.axis_index("c")
    sid = lax.axis_index("s")
    wid = sid * NC + cid
    pltpu.sync_copy(eid_hbm, eid_v)
    lanes = lax.iota(jnp.int32, 16)
    zero16 = jnp.zeros((16,), jnp.int32)

    def zero_body(i, c):
        tok_v[pl.ds(i * 16, 16)] = (i * 16 + lanes) & (T - 1)
        return c
    lax.fori_loop(0, P_ALLOC // 16, zero_body, 0)

    def hist_body(g, hist):
        v = eid_v[pl.ds(g * 16, 16)]
        for e in range(E):
            su = jnp.sum(jnp.where(v == e, 1, 0))
            hist = hist + jnp.where(lanes == e, su, 0)
        return hist
    hist = lax.fori_loop(0, A_N // 16, hist_body, zero16)

    npad = jnp.where(lanes < E,
                     ((hist + (PBLK - 1)) // PBLK) * PBLK, 0)
    incl = jnp.cumsum(npad)
    offs = incl - npad
    pt = jnp.sum(jnp.where(lanes < E, npad, 0))
    cnt0 = jnp.where(lanes < E, offs, 0)

    def scan_body(g, cnt):
        v = eid_v[pl.ds(g * 16, 16)]
        bases = cnt.at[v].get(mode="promise_in_bounds")
        r = zero16
        for e in range(E):
            mk = v == e
            mi = jnp.where(mk, 1, 0)
            c = jnp.cumsum(mi)
            r = jnp.where(mk, c - 1, r)
            cnt = cnt + jnp.where(lanes == e, jnp.sum(mi), 0)
        posv = jnp.where(v < E, bases + r,
                         DUMP + ((g * 16 + lanes) & (PBLK - 1)))
        tv = (g * 16 + lanes) >> 1
        pos_v[pl.ds(g * 16, 16)] = posv
        plsc.store_scatter(tok_v, (posv,), tv, mask=v < E)
        return cnt
    lax.fori_loop(0, A_N // 16, scan_body, cnt0)

    @pl.when(wid == 0)
    def _export():
        lastblk = jnp.maximum(pt // PBLK - 1, 0)
        exp0 = zero16
        exp1 = zero16
        lastexp = jnp.int32(0)
        row0 = lanes * PBLK
        row1 = (lanes + 16) * PBLK
        for e in range(E):
            se = jnp.sum(jnp.where(lanes == e, incl, 0))
            exp0 = exp0 + jnp.where(row0 >= se, 1, 0)
            exp1 = exp1 + jnp.where(row1 >= se, 1, 0)
            lastexp = lastexp + jnp.where(pt - PBLK >= se, 1, 0)
        act0 = jnp.where(row0 < pt, 1, 0)
        act1 = jnp.where(row1 < pt, 1, 0)
        meta_v[pl.ds(0, 16)] = jnp.where(act0 == 1, exp0, lastexp)
        meta_v[pl.ds(16, 16)] = jnp.where(act1 == 1, exp1, lastexp)
        meta_v[pl.ds(32, 16)] = act0
        meta_v[pl.ds(48, 16)] = act1
        meta_v[pl.ds(64, 16)] = (jnp.where(lanes == 0, lastblk, 0)
                                 + jnp.where(lanes == 1, pt, 0))
        meta_v[pl.ds(80, 16)] = zero16
        pltpu.sync_copy(pos_v, pos_hbm)
        pltpu.sync_copy(meta_v, meta_hbm)

    base = wid * XCH

    @pl.when(base < pt)
    def _gather():
        _pipe_gather(x_hbm, tok_v, base, xg_hbm, base, XCH // XSUB, XSUB,
                     (rows_a, rows_b), (gsem_a, gsem_b), (wsem_a, wsem_b))


def _pipe_gather(src_hbm, idx_v, ibase, out_hbm, obase, n, sub,
                 bufs, gsems, wsems):
    pltpu.async_copy(src_hbm.at[idx_v.at[pl.ds(ibase, sub)]],
                     bufs[0], gsems[0])
    for j in range(n):
        k = j % 2
        pltpu.make_async_copy(
            src_hbm.at[idx_v.at[pl.ds(ibase + j * sub, sub)]],
            bufs[k], gsems[k]).wait()
        if j + 1 < n:
            nk = (j + 1) % 2
            if j >= 1:
                pltpu.make_async_copy(bufs[nk], out_hbm.at[pl.ds(0, sub)],
                                      wsems[nk]).wait()
            pltpu.async_copy(
                src_hbm.at[idx_v.at[pl.ds(ibase + (j + 1) * sub, sub)]],
                bufs[nk], gsems[nk])
        pltpu.async_copy(bufs[k], out_hbm.at[pl.ds(obase + j * sub, sub)],
                         wsems[k])
    pltpu.make_async_copy(bufs[(n - 2) % 2], out_hbm.at[pl.ds(0, sub)],
                          wsems[(n - 2) % 2]).wait()
    pltpu.make_async_copy(bufs[(n - 1) % 2], out_hbm.at[pl.ds(0, sub)],
                          wsems[(n - 1) % 2]).wait()


def _group_body(meta_ref, xg_ref, w1_ref, w3_ref, w2_ref, yg_ref):
    b = pl.program_id(0)
    act = meta_ref[32 + b]

    @pl.when(act == 0)
    def _zero():
        yg_ref[...] = jnp.zeros((PBLK, H), jnp.float32)

    @pl.when(act == 1)
    def _compute():
        xb = xg_ref[...].astype(jnp.bfloat16)
        g = lax.dot_general(xb, w1_ref[0], (((1,), (1,)), ((), ())),
                            preferred_element_type=jnp.float32)
        u = lax.dot_general(xb, w3_ref[0], (((1,), (1,)), ((), ())),
                            preferred_element_type=jnp.float32)
        acts = ((g * jax.nn.sigmoid(g)) * u).astype(jnp.bfloat16)
        yg_ref[...] = lax.dot_general(
            acts, w2_ref[0], (((1,), (1,)), ((), ())),
            preferred_element_type=jnp.float32)


def _cgather_body(yg_hbm, pos_hbm, yt_hbm, idx_v, rows_a, rows_b,
                  gsem_a, gsem_b, wsem_a, wsem_b):
    wid = lax.axis_index("s") * NC + lax.axis_index("c")
    base = wid * DCH
    pltpu.sync_copy(pos_hbm.at[pl.ds(base, DCH)], idx_v)
    _pipe_gather(yg_hbm, idx_v, 0, yt_hbm, base, DCH // DSUB, DSUB,
                 (rows_a, rows_b), (gsem_a, gsem_b), (wsem_a, wsem_b))


def _final_body(x_ref, yt_ref, swt_ref, out_ref):
    x = x_ref[...]
    yt = yt_ref[...]
    swt = swt_ref[...]
    col = lax.broadcasted_iota(jnp.int32, (TBLK, LANES), 1)
    w0 = jnp.sum(jnp.where(col == 0, swt, 0.0), axis=1, keepdims=True)
    w1 = jnp.sum(jnp.where(col == 1, swt, 0.0), axis=1, keepdims=True)
    zw = jnp.sum(jnp.where(col == 2, swt, 0.0), axis=1, keepdims=True)
    out_ref[...] = zw * x + w0 * yt[:, :H] + w1 * yt[:, H:]


_SC_MESH = plsc.VectorSubcoreMesh(core_axis_name="c", subcore_axis_name="s")


@jax.jit
def kernel(hidden_states, router_weight, e_score_correction_bias, w1, w3, w2):
    x = hidden_states.astype(jnp.float32)
    rw = jnp.zeros((LANES, H), jnp.float32).at[:NE].set(router_weight)
    bias = jnp.full((1, LANES), NEG, jnp.float32).at[0, :NE].set(
        e_score_correction_bias)
    nt = T // TBLK

    sel, swt = pl.pallas_call(
        _router_body,
        grid=(nt,),
        in_specs=[
            pl.BlockSpec((TBLK, H), lambda t: (t, 0)),
            pl.BlockSpec((LANES, H), lambda t: (0, 0)),
            pl.BlockSpec((1, LANES), lambda t: (0, 0)),
        ],
        out_specs=[
            pl.BlockSpec((TBLK, LANES), lambda t: (t, 0)),
            pl.BlockSpec((TBLK, LANES), lambda t: (t, 0)),
        ],
        out_shape=[
            jax.ShapeDtypeStruct((T, LANES), jnp.int32),
            jax.ShapeDtypeStruct((T, LANES), jnp.float32),
        ],
    )(x, rw, bias)

    eid = sel[:, :K].reshape(A_N)

    dispgather = functools.partial(
        pl.kernel,
        compiler_params=pltpu.CompilerParams(needs_layout_passes=False),
        out_type=(
            jax.ShapeDtypeStruct((P_ALLOC, H), jnp.float32),
            jax.ShapeDtypeStruct((A_N,), jnp.int32),
            jax.ShapeDtypeStruct((96,), jnp.int32),
        ),
        mesh=_SC_MESH,
        scratch_types=[
            pltpu.VMEM((A_N,), jnp.int32),
            pltpu.VMEM((P_ALLOC,), jnp.int32),
            pltpu.VMEM((A_N,), jnp.int32),
            pltpu.VMEM((96,), jnp.int32),
            pltpu.VMEM((XSUB, H), jnp.float32),
            pltpu.VMEM((XSUB, H), jnp.float32),
            pltpu.SemaphoreType.DMA,
            pltpu.SemaphoreType.DMA,
            pltpu.SemaphoreType.DMA,
            pltpu.SemaphoreType.DMA,
        ],
    )(_dispgather_body)
    xg, pos, meta = dispgather(eid, x)

    yg = pl.pallas_call(
        _group_body,
        grid_spec=pltpu.PrefetchScalarGridSpec(
            num_scalar_prefetch=1,
            grid=(NBLK,),
            in_specs=[
                pl.BlockSpec((PBLK, H),
                             lambda b, m: (jnp.where(m[32 + b] == 1, b,
                                                     m[64]), 0)),
                pl.BlockSpec((1, I, H), lambda b, m: (m[b], 0, 0)),
                pl.BlockSpec((1, I, H), lambda b, m: (m[b], 0, 0)),
                pl.BlockSpec((1, H, I), lambda b, m: (m[b], 0, 0)),
            ],
            out_specs=pl.BlockSpec((PBLK, H), lambda b, m: (b, 0)),
        ),
        out_shape=jax.ShapeDtypeStruct((P_ALLOC, H), jnp.float32),
        compiler_params=pltpu.CompilerParams(
            dimension_semantics=("arbitrary",)),
    )(meta, xg, w1.astype(jnp.bfloat16), w3.astype(jnp.bfloat16),
      w2.astype(jnp.bfloat16))

    cgather = functools.partial(
        pl.kernel,
        compiler_params=pltpu.CompilerParams(needs_layout_passes=False),
        out_type=jax.ShapeDtypeStruct((A_N, H), jnp.float32),
        mesh=_SC_MESH,
        scratch_types=[
            pltpu.VMEM((DCH,), jnp.int32),
            pltpu.VMEM((DSUB, H), jnp.float32),
            pltpu.VMEM((DSUB, H), jnp.float32),
            pltpu.SemaphoreType.DMA,
            pltpu.SemaphoreType.DMA,
            pltpu.SemaphoreType.DMA,
            pltpu.SemaphoreType.DMA,
        ],
    )(_cgather_body)
    yt = cgather(yg, pos)

    out = pl.pallas_call(
        _final_body,
        grid=(nt,),
        in_specs=[
            pl.BlockSpec((TBLK, H), lambda t: (t, 0)),
            pl.BlockSpec((TBLK, 2 * H), lambda t: (t, 0)),
            pl.BlockSpec((TBLK, LANES), lambda t: (t, 0)),
        ],
        out_specs=pl.BlockSpec((TBLK, H), lambda t: (t, 0)),
        out_shape=jax.ShapeDtypeStruct((T, H), jnp.float32),
    )(x, yt.reshape(T, K * H), swt)
    return out
